# baseline (speedup 1.0000x reference)
import functools

import jax
import jax.numpy as jnp
from jax import lax
from jax.experimental import pallas as pl
from jax.experimental.pallas import tpu as pltpu
from jax.experimental.pallas import tpu_sc as plsc

N = 100000
E = 6400000

NC = 2
NS = 16
NW = NC * NS

CHUNK = 2048
SUB = 2048
NSUB = CHUNK // SUB
SUBG = SUB // 16
NCHUNKS = E // CHUNK
T_ITERS = (NCHUNKS + NW - 1) // NW
N_PAD = 100096
ROWS_PER_TILE = N_PAD // NS

_BETA = 0.3
_SIGMA = 1.0
_CUTOFF = 2.5


def _sc_partial_forces(px, py, pz, idx_i, idx_j):
    mesh = plsc.VectorSubcoreMesh(core_axis_name="c", subcore_axis_name="s")

    scratch = []
    for _ in range(3):
        scratch.append(pltpu.VMEM((NSUB, SUB), jnp.int32))
        scratch.append(pltpu.VMEM((NSUB, SUB), jnp.int32))
        for _ in range(12):
            scratch.append(pltpu.VMEM((NSUB, SUB), jnp.float32))
    scratch += [
        pltpu.VMEM((CHUNK,), jnp.float32),
        pltpu.VMEM((1, 16), jnp.float32),
        pltpu.VMEM((ROWS_PER_TILE,), jnp.float32),
        pltpu.VMEM_SHARED((N_PAD,), jnp.float32),
        pltpu.VMEM_SHARED((N_PAD,), jnp.float32),
        pltpu.VMEM_SHARED((N_PAD,), jnp.float32),
    ]
    scratch += [pltpu.SemaphoreType.DMA] * 3
    scratch += [pltpu.SemaphoreType.DMA] * 3

    @functools.partial(
        pl.kernel,
        mesh=mesh,
        out_type=jax.ShapeDtypeStruct((NC * 3 * N_PAD,), jnp.float32),
        scratch_types=scratch,
    )
    def k(i3_hbm, j3_hbm, px_hbm, py_hbm, pz_hbm, out_hbm, *scr):
        buf = [scr[14 * b:14 * (b + 1)] for b in range(3)]
        zb_v, rs_v, ob_v, accx, accy, accz = scr[42:48]
        gsem = scr[48:51]
        ssem = scr[51:54]

        c = lax.axis_index("c")
        s = lax.axis_index("s")
        wid = c * NS + s

        zf = jnp.zeros((16,), jnp.float32)
        rs_i = rs_v.bitcast(jnp.int32)

        def gather_list(b):
            ii_v, jj_v = buf[b][0], buf[b][1]
            planes = buf[b][2:8]
            tbls = (px_hbm, py_hbm, pz_hbm, px_hbm, py_hbm, pz_hbm)
            idxs = (ii_v, ii_v, ii_v, jj_v, jj_v, jj_v)
            out = []
            for r in range(NSUB):
                for tbl, idx, dst in zip(tbls, idxs, planes):
                    out.append((tbl.at[idx.at[r]], dst.at[r], gsem[b]))
            return out

        def scatter_list(b):
            ii_v, jj_v = buf[b][0], buf[b][1]
            fvs = buf[b][8:14]
            accs = (accx, accy, accz, accx, accy, accz)
            idxs = (ii_v, ii_v, ii_v, jj_v, jj_v, jj_v)
            out = []
            for r in range(NSUB):
                for src, idx, acc in zip(fvs, idxs, accs):
                    out.append((src.at[r], acc.at[idx.at[r]], ssem[b]))
            return out

        def stage_and_fire(b, k_idx):
            pltpu.sync_copy(i3_hbm.at[k_idx], buf[b][0])
            pltpu.sync_copy(j3_hbm.at[k_idx], buf[b][1])
            for src, dst, sem in gather_list(b):
                pltpu.async_copy(src, dst, sem)

        def drain_gathers(b):
            for src, dst, sem in gather_list(b):
                pltpu.make_async_copy(src, dst, sem).wait()

        def fire_scatters(b):
            for src, dst, sem in scatter_list(b):
                pltpu.async_copy(src, dst, sem, add=True)

        def drain_scatters(b):
            for src, dst, sem in scatter_list(b):
                pltpu.make_async_copy(src, dst, sem).wait()

        def compute(b):
            xi_v, yi_v, zi_v, xj_v, yj_v, zj_v = buf[b][2:8]
            fix_v, fiy_v, fiz_v, fjx_v, fjy_v, fjz_v = buf[b][8:14]

            def comp_body(t, _):
                r = lax.div(t, SUBG)
                o = lax.rem(t, SUBG) * 16
                xi = xi_v[r, pl.ds(o, 16)]
                yi = yi_v[r, pl.ds(o, 16)]
                zi = zi_v[r, pl.ds(o, 16)]
                xj = xj_v[r, pl.ds(o, 16)]
                yj = yj_v[r, pl.ds(o, 16)]
                zj = zj_v[r, pl.ds(o, 16)]
                dx = xj - xi
                dy = yj - yi
                dz = zj - zi
                d2 = dx * dx + dy * dy + dz * dz
                d2s = jnp.where(d2 > 0.0, d2, jnp.full((16,), 1.0, jnp.float32))
                rs_v[0, pl.ds(0, 16)] = d2s
                bits = rs_i[0, pl.ds(0, 16)]
                rs_i[0, pl.ds(0, 16)] = jnp.full(
                    (16,), 0x5F3759DF, jnp.int32
                ) - lax.shift_right_logical(
                    bits, jnp.full((16,), 1, jnp.int32)
                )
                r_inv = rs_v[0, pl.ds(0, 16)]
                half = d2s * 0.5
                for _ in range(3):
                    r_inv = r_inv * (1.5 - half * r_inv * r_inv)
                d = d2 * r_inv
                inner = jnp.where(d < _BETA, d * (1.0 / _BETA) - 1.0, zf)
                outer = jnp.where(
                    (d < _SIGMA) & (d > _BETA),
                    1.0 - jnp.abs(2.0 * d - 1.0 - _BETA) * (1.0 / (1.0 - _BETA)),
                    zf,
                )
                f = inner + outer
                f = jnp.where(d < _CUTOFF, f, zf)
                scale = f * r_inv
                fvx = scale * dx
                fvy = scale * dy
                fvz = scale * dz
                fjx_v[r, pl.ds(o, 16)] = fvx
                fjy_v[r, pl.ds(o, 16)] = fvy
                fjz_v[r, pl.ds(o, 16)] = fvz
                fix_v[r, pl.ds(o, 16)] = -fvx
                fiy_v[r, pl.ds(o, 16)] = -fvy
                fiz_v[r, pl.ds(o, 16)] = -fvz
                return 0

            lax.fori_loop(0, CHUNK // 16, comp_body, 0)

        def zero_body(t, _):
            zb_v[pl.ds(t * 16, 16)] = zf
            return 0

        lax.fori_loop(0, CHUNK // 16, zero_body, 0)

        base = s * ROWS_PER_TILE
        for acc in (accx, accy, accz):
            pltpu.sync_copy(zb_v, acc.at[pl.ds(base, CHUNK)])
            pltpu.sync_copy(zb_v, acc.at[pl.ds(base + CHUNK, CHUNK)])
            pltpu.sync_copy(zb_v, acc.at[pl.ds(base + 2 * CHUNK, CHUNK)])
            pltpu.sync_copy(
                zb_v.at[pl.ds(0, ROWS_PER_TILE - 3 * CHUNK)],
                acc.at[pl.ds(base + 3 * CHUNK, ROWS_PER_TILE - 3 * CHUNK)],
            )
        plsc.subcore_barrier()

        @pl.when(wid < NCHUNKS)
        def _():
            stage_and_fire(0, wid)

        def triple_body(tp, _):
            for b in (0, 1, 2):
                t = tp * 3 + b
                k_cur = wid + t * NW
                k_nxt = k_cur + NW
                nb = (b + 1) % 3

                if b == 2:
                    @pl.when(k_cur - 2 * NW < NCHUNKS)
                    def _():
                        drain_scatters(nb)
                else:
                    @pl.when((tp > 0) & (k_cur - 2 * NW < NCHUNKS))
                    def _():
                        drain_scatters(nb)

                @pl.when(k_nxt < NCHUNKS)
                def _():
                    stage_and_fire(nb, k_nxt)

                @pl.when(k_cur < NCHUNKS)
                def _():
                    drain_gathers(b)
                    compute(b)
                    fire_scatters(b)
            return 0

        lax.fori_loop(0, (T_ITERS + 1 + 2) // 3, triple_body, 0)

        @pl.when(wid + (T_ITERS - 1) * NW < NCHUNKS)
        def _():
            drain_scatters((T_ITERS - 1) % 3)

        plsc.subcore_barrier()
        obase = c * (3 * N_PAD) + base
        for q, acc in enumerate((accx, accy, accz)):
            pltpu.sync_copy(acc.at[pl.ds(base, ROWS_PER_TILE)], ob_v)
            pltpu.sync_copy(
                ob_v, out_hbm.at[pl.ds(obase + q * N_PAD, ROWS_PER_TILE)]
            )

    return k(idx_i, idx_j, px, py, pz)


def _tc_sum(a_ref, o_ref):
    o_ref[...] = a_ref[0] + a_ref[1]


def kernel(positions, mapping):
    px = jnp.pad(positions[:, 0], (0, N_PAD - N))
    py = jnp.pad(positions[:, 1], (0, N_PAD - N))
    pz = jnp.pad(positions[:, 2], (0, N_PAD - N))
    idx_i = mapping[0].reshape(NCHUNKS, NSUB, SUB)
    idx_j = mapping[1].reshape(NCHUNKS, NSUB, SUB)
    partials = _sc_partial_forces(px, py, pz, idx_i, idx_j)
    summed = pl.pallas_call(
        _tc_sum,
        out_shape=jax.ShapeDtypeStruct((3 * N_PAD // 128, 128), jnp.float32),
    )(partials.reshape(NC, 3 * N_PAD // 128, 128))
    return summed.reshape(3, N_PAD)[:, :N].T


# speedup vs baseline: 1.0120x; 1.0120x over previous
import functools

import jax
import jax.numpy as jnp
from jax import lax
from jax.experimental import pallas as pl
from jax.experimental.pallas import tpu as pltpu
from jax.experimental.pallas import tpu_sc as plsc

N = 100000
E = 6400000

NC = 2
NS = 16
NW = NC * NS

CHUNK = 1024
SUB = 128
NSUB = CHUNK // SUB
SUBG = SUB // 16
NCHUNKS = E // CHUNK
T_ITERS = (NCHUNKS + NW - 1) // NW
ZB = 2048
N_PAD = 100096
ROWS_PER_TILE = N_PAD // NS

_BETA = 0.3
_SIGMA = 1.0
_CUTOFF = 2.5


def _sc_partial_forces(px, py, pz, idx_i, idx_j):
    mesh = plsc.VectorSubcoreMesh(core_axis_name="c", subcore_axis_name="s")

    scratch = []
    for _ in range(4):
        scratch.append(pltpu.VMEM((NSUB, SUB), jnp.int32))
        scratch.append(pltpu.VMEM((NSUB, SUB), jnp.int32))
        for _ in range(12):
            scratch.append(pltpu.VMEM((NSUB, SUB), jnp.float32))
    scratch += [
        pltpu.VMEM((ZB,), jnp.float32),
        pltpu.VMEM((1, 16), jnp.float32),
        pltpu.VMEM((ZB,), jnp.float32),
        pltpu.VMEM_SHARED((N_PAD,), jnp.float32),
        pltpu.VMEM_SHARED((N_PAD,), jnp.float32),
        pltpu.VMEM_SHARED((N_PAD,), jnp.float32),
    ]
    scratch += [pltpu.SemaphoreType.DMA] * 4
    scratch += [pltpu.SemaphoreType.DMA] * 4
    scratch += [pltpu.SemaphoreType.DMA] * 4

    @functools.partial(
        pl.kernel,
        mesh=mesh,
        out_type=jax.ShapeDtypeStruct((NC * 3 * N_PAD,), jnp.float32),
        scratch_types=scratch,
    )
    def k(i3_hbm, j3_hbm, px_hbm, py_hbm, pz_hbm, out_hbm, *scr):
        buf = [scr[14 * b:14 * (b + 1)] for b in range(4)]
        zb_v, rs_v, ob_v, accx, accy, accz = scr[56:62]
        gsem = scr[62:66]
        ssem = scr[66:70]
        isem = scr[70:74]

        c = lax.axis_index("c")
        s = lax.axis_index("s")
        wid = c * NS + s

        zf = jnp.zeros((16,), jnp.float32)
        rs_i = rs_v.bitcast(jnp.int32)

        def gather_list(b):
            ii_v, jj_v = buf[b][0], buf[b][1]
            planes = buf[b][2:8]
            tbls = (px_hbm, py_hbm, pz_hbm, px_hbm, py_hbm, pz_hbm)
            idxs = (ii_v, ii_v, ii_v, jj_v, jj_v, jj_v)
            out = []
            for r in range(NSUB):
                for tbl, idx, dst in zip(tbls, idxs, planes):
                    out.append((tbl.at[idx.at[r]], dst.at[r], gsem[b]))
            return out

        def scatter_list(b):
            ii_v, jj_v = buf[b][0], buf[b][1]
            fvs = buf[b][8:14]
            accs = (accx, accy, accz, accx, accy, accz)
            idxs = (ii_v, ii_v, ii_v, jj_v, jj_v, jj_v)
            out = []
            for r in range(NSUB):
                for src, idx, acc in zip(fvs, idxs, accs):
                    out.append((src.at[r], acc.at[idx.at[r]], ssem[b]))
            return out

        def stage_idx(b, k_idx):
            pltpu.async_copy(i3_hbm.at[k_idx], buf[b][0], isem[b])
            pltpu.async_copy(j3_hbm.at[k_idx], buf[b][1], isem[b])

        def wait_idx_and_fire(b, k_idx):
            pltpu.make_async_copy(i3_hbm.at[k_idx], buf[b][0], isem[b]).wait()
            pltpu.make_async_copy(j3_hbm.at[k_idx], buf[b][1], isem[b]).wait()
            for src, dst, sem in gather_list(b):
                pltpu.async_copy(src, dst, sem)

        def drain_gathers(b):
            for src, dst, sem in gather_list(b):
                pltpu.make_async_copy(src, dst, sem).wait()

        def fire_scatters(b):
            for src, dst, sem in scatter_list(b):
                pltpu.async_copy(src, dst, sem, add=True)

        def drain_scatters(b):
            for src, dst, sem in scatter_list(b):
                pltpu.make_async_copy(src, dst, sem).wait()

        def compute(b):
            xi_v, yi_v, zi_v, xj_v, yj_v, zj_v = buf[b][2:8]
            fix_v, fiy_v, fiz_v, fjx_v, fjy_v, fjz_v = buf[b][8:14]

            def comp_body(t, _):
                r = lax.div(t, SUBG)
                o = lax.rem(t, SUBG) * 16
                xi = xi_v[r, pl.ds(o, 16)]
                yi = yi_v[r, pl.ds(o, 16)]
                zi = zi_v[r, pl.ds(o, 16)]
                xj = xj_v[r, pl.ds(o, 16)]
                yj = yj_v[r, pl.ds(o, 16)]
                zj = zj_v[r, pl.ds(o, 16)]
                dx = xj - xi
                dy = yj - yi
                dz = zj - zi
                d2 = dx * dx + dy * dy + dz * dz
                d2s = jnp.where(d2 > 0.0, d2, jnp.full((16,), 1.0, jnp.float32))
                rs_v[0, pl.ds(0, 16)] = d2s
                bits = rs_i[0, pl.ds(0, 16)]
                rs_i[0, pl.ds(0, 16)] = jnp.full(
                    (16,), 0x5F3759DF, jnp.int32
                ) - lax.shift_right_logical(
                    bits, jnp.full((16,), 1, jnp.int32)
                )
                r_inv = rs_v[0, pl.ds(0, 16)]
                half = d2s * 0.5
                for _ in range(3):
                    r_inv = r_inv * (1.5 - half * r_inv * r_inv)
                d = d2 * r_inv
                inner = jnp.where(d < _BETA, d * (1.0 / _BETA) - 1.0, zf)
                outer = jnp.where(
                    (d < _SIGMA) & (d > _BETA),
                    1.0 - jnp.abs(2.0 * d - 1.0 - _BETA) * (1.0 / (1.0 - _BETA)),
                    zf,
                )
                f = inner + outer
                f = jnp.where(d < _CUTOFF, f, zf)
                scale = f * r_inv
                fvx = scale * dx
                fvy = scale * dy
                fvz = scale * dz
                fjx_v[r, pl.ds(o, 16)] = fvx
                fjy_v[r, pl.ds(o, 16)] = fvy
                fjz_v[r, pl.ds(o, 16)] = fvz
                fix_v[r, pl.ds(o, 16)] = -fvx
                fiy_v[r, pl.ds(o, 16)] = -fvy
                fiz_v[r, pl.ds(o, 16)] = -fvz
                return 0

            lax.fori_loop(0, CHUNK // 16, comp_body, 0)

        def zero_body(t, _):
            zb_v[pl.ds(t * 16, 16)] = zf
            return 0

        lax.fori_loop(0, ZB // 16, zero_body, 0)

        base = s * ROWS_PER_TILE
        for acc in (accx, accy, accz):
            pltpu.sync_copy(zb_v, acc.at[pl.ds(base, ZB)])
            pltpu.sync_copy(zb_v, acc.at[pl.ds(base + ZB, ZB)])
            pltpu.sync_copy(zb_v, acc.at[pl.ds(base + 2 * ZB, ZB)])
            pltpu.sync_copy(
                zb_v.at[pl.ds(0, ROWS_PER_TILE - 3 * ZB)],
                acc.at[pl.ds(base + 3 * ZB, ROWS_PER_TILE - 3 * ZB)],
            )
        plsc.subcore_barrier()

        stage_idx(0, wid)

        @pl.when(wid + NW < NCHUNKS)
        def _():
            stage_idx(1, wid + NW)

        wait_idx_and_fire(0, wid)

        def quad_body(tp, _):
            for b in (0, 1, 2, 3):
                t = tp * 4 + b
                k_cur = wid + t * NW
                k_nxt = k_cur + NW
                k_nxt2 = k_cur + 2 * NW
                nb = (b + 1) % 4
                nb2 = (b + 2) % 4

                if b >= 2:
                    @pl.when(k_cur - 2 * NW < NCHUNKS)
                    def _():
                        drain_scatters(nb2)
                else:
                    @pl.when((tp > 0) & (k_cur - 2 * NW < NCHUNKS))
                    def _():
                        drain_scatters(nb2)

                @pl.when(k_nxt2 < NCHUNKS)
                def _():
                    stage_idx(nb2, k_nxt2)

                @pl.when(k_nxt < NCHUNKS)
                def _():
                    wait_idx_and_fire(nb, k_nxt)

                @pl.when(k_cur < NCHUNKS)
                def _():
                    drain_gathers(b)
                    compute(b)
                    fire_scatters(b)
            return 0

        lax.fori_loop(0, (T_ITERS + 2 + 3) // 4, quad_body, 0)

        plsc.subcore_barrier()
        obase = c * (3 * N_PAD) + base
        tail = ROWS_PER_TILE - 3 * ZB
        for q, acc in enumerate((accx, accy, accz)):
            for p in range(3):
                pltpu.sync_copy(acc.at[pl.ds(base + p * ZB, ZB)], ob_v)
                pltpu.sync_copy(
                    ob_v,
                    out_hbm.at[pl.ds(obase + q * N_PAD + p * ZB, ZB)],
                )
            pltpu.sync_copy(
                acc.at[pl.ds(base + 3 * ZB, tail)], ob_v.at[pl.ds(0, tail)]
            )
            pltpu.sync_copy(
                ob_v.at[pl.ds(0, tail)],
                out_hbm.at[pl.ds(obase + q * N_PAD + 3 * ZB, tail)],
            )

    return k(idx_i, idx_j, px, py, pz)


def _tc_sum(a_ref, o_ref):
    o_ref[...] = a_ref[0] + a_ref[1]


def kernel(positions, mapping):
    px = jnp.pad(positions[:, 0], (0, N_PAD - N))
    py = jnp.pad(positions[:, 1], (0, N_PAD - N))
    pz = jnp.pad(positions[:, 2], (0, N_PAD - N))
    idx_i = mapping[0].reshape(NCHUNKS, NSUB, SUB)
    idx_j = mapping[1].reshape(NCHUNKS, NSUB, SUB)
    partials = _sc_partial_forces(px, py, pz, idx_i, idx_j)
    summed = pl.pallas_call(
        _tc_sum,
        out_shape=jax.ShapeDtypeStruct((3 * N_PAD // 128, 128), jnp.float32),
    )(partials.reshape(NC, 3 * N_PAD // 128, 128))
    return summed.reshape(3, N_PAD)[:, :N].T


# speedup vs baseline: 1.0287x; 1.0165x over previous
import functools

import jax
import jax.numpy as jnp
from jax import lax
from jax.experimental import pallas as pl
from jax.experimental.pallas import tpu as pltpu
from jax.experimental.pallas import tpu_sc as plsc

N = 100000
E = 6400000

NC = 2
NS = 16
NW = NC * NS

CHUNK = 2048
SUB = 128
NSUB = CHUNK // SUB
SUBG = SUB // 16
NCHUNKS = E // CHUNK
T_ITERS = (NCHUNKS + NW - 1) // NW
ZB = 2048
N_PAD = 100096
ROWS_PER_TILE = N_PAD // NS

_BETA = 0.3
_SIGMA = 1.0
_CUTOFF = 2.5


def _sc_partial_forces(px, py, pz, idx_i, idx_j):
    mesh = plsc.VectorSubcoreMesh(core_axis_name="c", subcore_axis_name="s")

    scratch = []
    for _ in range(3):
        scratch.append(pltpu.VMEM((NSUB, SUB), jnp.int32))
        scratch.append(pltpu.VMEM((NSUB, SUB), jnp.int32))
        for _ in range(12):
            scratch.append(pltpu.VMEM((NSUB, SUB), jnp.float32))
    scratch += [
        pltpu.VMEM((ZB,), jnp.float32),
        pltpu.VMEM((1, 16), jnp.float32),
        pltpu.VMEM((ZB,), jnp.float32),
        pltpu.VMEM_SHARED((N_PAD,), jnp.float32),
        pltpu.VMEM_SHARED((N_PAD,), jnp.float32),
        pltpu.VMEM_SHARED((N_PAD,), jnp.float32),
    ]
    scratch += [pltpu.SemaphoreType.DMA] * 3
    scratch += [pltpu.SemaphoreType.DMA] * 3

    @functools.partial(
        pl.kernel,
        mesh=mesh,
        out_type=jax.ShapeDtypeStruct((NC * 3 * N_PAD,), jnp.float32),
        scratch_types=scratch,
    )
    def k(i3_hbm, j3_hbm, px_hbm, py_hbm, pz_hbm, out_hbm, *scr):
        buf = [scr[14 * b:14 * (b + 1)] for b in range(3)]
        zb_v, rs_v, ob_v, accx, accy, accz = scr[42:48]
        gsem = scr[48:51]
        ssem = scr[51:54]

        c = lax.axis_index("c")
        s = lax.axis_index("s")
        wid = c * NS + s

        zf = jnp.zeros((16,), jnp.float32)
        rs_i = rs_v.bitcast(jnp.int32)

        def gather_list(b):
            ii_v, jj_v = buf[b][0], buf[b][1]
            planes = buf[b][2:8]
            tbls = (px_hbm, py_hbm, pz_hbm, px_hbm, py_hbm, pz_hbm)
            idxs = (ii_v, ii_v, ii_v, jj_v, jj_v, jj_v)
            out = []
            for r in range(NSUB):
                for tbl, idx, dst in zip(tbls, idxs, planes):
                    out.append((tbl.at[idx.at[r]], dst.at[r], gsem[b]))
            return out

        def scatter_list(b):
            ii_v, jj_v = buf[b][0], buf[b][1]
            fvs = buf[b][8:14]
            accs = (accx, accy, accz, accx, accy, accz)
            idxs = (ii_v, ii_v, ii_v, jj_v, jj_v, jj_v)
            out = []
            for r in range(NSUB):
                for src, idx, acc in zip(fvs, idxs, accs):
                    out.append((src.at[r], acc.at[idx.at[r]], ssem[b]))
            return out

        def stage_and_fire(b, k_idx):
            pltpu.sync_copy(i3_hbm.at[k_idx], buf[b][0])
            pltpu.sync_copy(j3_hbm.at[k_idx], buf[b][1])
            for src, dst, sem in gather_list(b):
                pltpu.async_copy(src, dst, sem)

        def drain_gathers(b):
            for src, dst, sem in gather_list(b):
                pltpu.make_async_copy(src, dst, sem).wait()

        def fire_scatters(b):
            for src, dst, sem in scatter_list(b):
                pltpu.async_copy(src, dst, sem, add=True)

        def drain_scatters(b):
            for src, dst, sem in scatter_list(b):
                pltpu.make_async_copy(src, dst, sem).wait()

        def compute(b):
            xi_v, yi_v, zi_v, xj_v, yj_v, zj_v = buf[b][2:8]
            fix_v, fiy_v, fiz_v, fjx_v, fjy_v, fjz_v = buf[b][8:14]

            def comp_body(t, _):
                r = lax.div(t, SUBG)
                o = lax.rem(t, SUBG) * 16
                xi = xi_v[r, pl.ds(o, 16)]
                yi = yi_v[r, pl.ds(o, 16)]
                zi = zi_v[r, pl.ds(o, 16)]
                xj = xj_v[r, pl.ds(o, 16)]
                yj = yj_v[r, pl.ds(o, 16)]
                zj = zj_v[r, pl.ds(o, 16)]
                dx = xj - xi
                dy = yj - yi
                dz = zj - zi
                d2 = dx * dx + dy * dy + dz * dz
                d2s = jnp.where(d2 > 0.0, d2, jnp.full((16,), 1.0, jnp.float32))
                rs_v[0, pl.ds(0, 16)] = d2s
                bits = rs_i[0, pl.ds(0, 16)]
                rs_i[0, pl.ds(0, 16)] = jnp.full(
                    (16,), 0x5F3759DF, jnp.int32
                ) - lax.shift_right_logical(
                    bits, jnp.full((16,), 1, jnp.int32)
                )
                r_inv = rs_v[0, pl.ds(0, 16)]
                half = d2s * 0.5
                for _ in range(3):
                    r_inv = r_inv * (1.5 - half * r_inv * r_inv)
                d = d2 * r_inv
                inner = jnp.where(d < _BETA, d * (1.0 / _BETA) - 1.0, zf)
                outer = jnp.where(
                    (d < _SIGMA) & (d > _BETA),
                    1.0 - jnp.abs(2.0 * d - 1.0 - _BETA) * (1.0 / (1.0 - _BETA)),
                    zf,
                )
                f = inner + outer
                f = jnp.where(d < _CUTOFF, f, zf)
                scale = f * r_inv
                fvx = scale * dx
                fvy = scale * dy
                fvz = scale * dz
                fjx_v[r, pl.ds(o, 16)] = fvx
                fjy_v[r, pl.ds(o, 16)] = fvy
                fjz_v[r, pl.ds(o, 16)] = fvz
                fix_v[r, pl.ds(o, 16)] = -fvx
                fiy_v[r, pl.ds(o, 16)] = -fvy
                fiz_v[r, pl.ds(o, 16)] = -fvz
                return 0

            lax.fori_loop(0, CHUNK // 16, comp_body, 0)

        def zero_body(t, _):
            zb_v[pl.ds(t * 16, 16)] = zf
            return 0

        lax.fori_loop(0, ZB // 16, zero_body, 0)

        base = s * ROWS_PER_TILE
        for acc in (accx, accy, accz):
            pltpu.sync_copy(zb_v, acc.at[pl.ds(base, ZB)])
            pltpu.sync_copy(zb_v, acc.at[pl.ds(base + ZB, ZB)])
            pltpu.sync_copy(zb_v, acc.at[pl.ds(base + 2 * ZB, ZB)])
            pltpu.sync_copy(
                zb_v.at[pl.ds(0, ROWS_PER_TILE - 3 * ZB)],
                acc.at[pl.ds(base + 3 * ZB, ROWS_PER_TILE - 3 * ZB)],
            )
        plsc.subcore_barrier()

        @pl.when(wid < NCHUNKS)
        def _():
            stage_and_fire(0, wid)

        def triple_body(tp, _):
            for b in (0, 1, 2):
                t = tp * 3 + b
                k_cur = wid + t * NW
                k_nxt = k_cur + NW
                nb = (b + 1) % 3

                if b == 2:
                    @pl.when(k_cur - 2 * NW < NCHUNKS)
                    def _():
                        drain_scatters(nb)
                else:
                    @pl.when((tp > 0) & (k_cur - 2 * NW < NCHUNKS))
                    def _():
                        drain_scatters(nb)

                @pl.when(k_nxt < NCHUNKS)
                def _():
                    stage_and_fire(nb, k_nxt)

                @pl.when(k_cur < NCHUNKS)
                def _():
                    drain_gathers(b)
                    compute(b)
                    fire_scatters(b)
            return 0

        lax.fori_loop(0, (T_ITERS + 1 + 2) // 3, triple_body, 0)

        @pl.when(wid + (T_ITERS - 1) * NW < NCHUNKS)
        def _():
            drain_scatters((T_ITERS - 1) % 3)

        plsc.subcore_barrier()
        obase = c * (3 * N_PAD) + base
        tail = ROWS_PER_TILE - 3 * ZB
        for q, acc in enumerate((accx, accy, accz)):
            for p in range(3):
                pltpu.sync_copy(acc.at[pl.ds(base + p * ZB, ZB)], ob_v)
                pltpu.sync_copy(
                    ob_v,
                    out_hbm.at[pl.ds(obase + q * N_PAD + p * ZB, ZB)],
                )
            pltpu.sync_copy(
                acc.at[pl.ds(base + 3 * ZB, tail)], ob_v.at[pl.ds(0, tail)]
            )
            pltpu.sync_copy(
                ob_v.at[pl.ds(0, tail)],
                out_hbm.at[pl.ds(obase + q * N_PAD + 3 * ZB, tail)],
            )

    return k(idx_i, idx_j, px, py, pz)


def _tc_sum(a_ref, o_ref):
    o_ref[...] = a_ref[0] + a_ref[1]


def kernel(positions, mapping):
    px = jnp.pad(positions[:, 0], (0, N_PAD - N))
    py = jnp.pad(positions[:, 1], (0, N_PAD - N))
    pz = jnp.pad(positions[:, 2], (0, N_PAD - N))
    idx_i = mapping[0].reshape(NCHUNKS, NSUB, SUB)
    idx_j = mapping[1].reshape(NCHUNKS, NSUB, SUB)
    partials = _sc_partial_forces(px, py, pz, idx_i, idx_j)
    summed = pl.pallas_call(
        _tc_sum,
        out_shape=jax.ShapeDtypeStruct((3 * N_PAD // 128, 128), jnp.float32),
    )(partials.reshape(NC, 3 * N_PAD // 128, 128))
    return summed.reshape(3, N_PAD)[:, :N].T
